# baseline (speedup 1.0000x reference)
import jax
import jax.numpy as jnp
from jax import lax
from jax.experimental import pallas as pl
from jax.experimental.pallas import tpu as pltpu
from jax.experimental.pallas import tpu_sc as plsc

PADDING_IDX = 0
D = 32
LANES = 16
NC, NS = 2, 16
NW = NC * NS
IDX_MINOR = 128
CHUNK = 1280
VREGS = CHUNK // LANES


def _zero_pad_rows(idx_v, rows_v, k):
    iv = idx_v[pl.ds(k * LANES, LANES)]
    m = iv == PADDING_IDX

    @pl.when(jnp.any(m))
    def _():
        row_ix = jnp.arange(LANES, dtype=jnp.int32) + k * LANES
        zeros = jnp.zeros((LANES,), dtype=jnp.float32)

        def col(j, carry):
            col_ix = jnp.full((LANES,), j, dtype=jnp.int32)
            plsc.store_scatter(rows_v, [row_ix, col_ix], zeros, mask=m)
            return carry

        lax.fori_loop(0, D, col, 0)


def _mask_fixup(idx_v, rows_v):
    def acc_any(k, m):
        return m | (idx_v[pl.ds(k * LANES, LANES)] == PADDING_IDX)

    m0 = jnp.zeros((LANES,), dtype=jnp.bool_)
    mall = lax.fori_loop(0, VREGS, acc_any, m0)

    @pl.when(jnp.any(mall))
    def _():
        def per_k(k, carry):
            _zero_pad_rows(idx_v, rows_v, k)
            return carry

        lax.fori_loop(0, VREGS, per_k, 0)


def _conv_body(embt_hbm, out_hbm, src_a, src_b, dst_a, dst_b, sem_a, sem_b):
    wid = lax.axis_index("s") * NC + lax.axis_index("c")
    n_full = 7812
    nbw = 244 + jnp.where(wid < n_full - 244 * NW, 1, 0)

    iota = jnp.arange(LANES, dtype=jnp.int32)

    def transpose_block(src, dst, ncols):
        def per_c(c, carry):
            for j0 in (0, 16):
                v = plsc.load_gather(src, [iota + j0,
                                           jnp.full((LANES,), c, jnp.int32)])
                flat = c * D + j0
                a = flat // 1024
                b = (flat % 1024) // IDX_MINOR
                off = flat % IDX_MINOR
                plsc.store_scatter(
                    dst,
                    [jnp.full((LANES,), a, jnp.int32),
                     jnp.full((LANES,), b, jnp.int32),
                     iota + off],
                    v)
            return carry

        lax.fori_loop(0, ncols, per_c, 0)

    def stage(i, src, sem):
        bid = wid + i * NW
        c0 = pl.multiple_of(bid * IDX_MINOR, IDX_MINOR)
        cp = pltpu.async_copy(
            embt_hbm.at[:, pl.ds(c0, IDX_MINOR)], src, sem)
        return cp

    def drain(i, src, dst, cp):
        bid = wid + i * NW
        cp.wait()
        transpose_block(src, dst, IDX_MINOR)
        pltpu.sync_copy(dst, out_hbm.at[pl.ds(bid * 4, 4)])

    def pair_body(p, carry):
        g = p * 2
        cpa = stage(g, src_a, sem_a)
        cpb = stage(g + 1, src_b, sem_b)
        drain(g, src_a, dst_a, cpa)
        drain(g + 1, src_b, dst_b, cpb)
        return carry

    lax.fori_loop(0, nbw // 2, pair_body, 0)

    @pl.when(nbw % 2 == 1)
    def _():
        g = nbw - 1
        cpa = stage(g, src_a, sem_a)
        drain(g, src_a, dst_a, cpa)

    @pl.when(wid == NW - 1)
    def _():
        c0 = n_full * IDX_MINOR

        def scoped(src_t, dst_t):
            cp = pltpu.async_copy(embt_hbm.at[:, pl.ds(c0, 64)], src_t, sem_a)
            cp.wait()
            transpose_block(src_t, dst_t, 64)
            pltpu.sync_copy(dst_t, out_hbm.at[pl.ds(c0 * D // 1024, 2)])

        pl.run_scoped(scoped,
                      pltpu.VMEM((D, 64), jnp.float32),
                      pltpu.VMEM((2, 8, IDX_MINOR), jnp.float32))


def _convert(embeddings):
    embt = embeddings.T
    mesh = plsc.VectorSubcoreMesh(core_axis_name="c", subcore_axis_name="s")
    k = pl.kernel(
        _conv_body,
        out_type=jax.ShapeDtypeStruct((31250, 8, IDX_MINOR), jnp.float32),
        mesh=mesh,
        scratch_types=[
            pltpu.VMEM((D, IDX_MINOR), jnp.float32),
            pltpu.VMEM((D, IDX_MINOR), jnp.float32),
            pltpu.VMEM((4, 8, IDX_MINOR), jnp.float32),
            pltpu.VMEM((4, 8, IDX_MINOR), jnp.float32),
            pltpu.SemaphoreType.DMA,
            pltpu.SemaphoreType.DMA,
        ],
        compiler_params=pltpu.CompilerParams(needs_layout_passes=False,
                                             use_tc_tiling_on_sc=True),
    )
    return k(embt)


def _sc_body(table_hbm, idx_hbm, out_hbm,
             idx_a, idx_b, rows_a, rows_b, sem_a, sem_b):
    wid = lax.axis_index("s") * NC + lax.axis_index("c")
    per_w = idx_hbm.shape[0] // NW
    base = wid * per_w
    n_pairs = per_w // (2 * CHUNK)

    def stage(g, idx_v, rows_v, sem):
        off = base + g * CHUNK
        pltpu.sync_copy(idx_hbm.at[pl.ds(off, CHUNK)], idx_v)
        return pltpu.async_copy(table_hbm.at[idx_v], rows_v, sem)

    def drain(g, idx_v, rows_v, cp):
        cp.wait()
        _mask_fixup(idx_v, rows_v)
        pltpu.sync_copy(rows_v, out_hbm.at[pl.ds(base + g * CHUNK, CHUNK)])

    def pair_body(p, carry):
        g = p * 2
        cpa = stage(g, idx_a, rows_a, sem_a)
        cpb = stage(g + 1, idx_b, rows_b, sem_b)
        drain(g, idx_a, rows_a, cpa)
        drain(g + 1, idx_b, rows_b, cpb)
        return carry

    lax.fori_loop(0, n_pairs, pair_body, 0)


def kernel(embeddings, inputs):
    B, L = inputs.shape
    total = B * L
    idx1d = inputs.reshape(total).astype(jnp.int32)

    table_rm = _convert(embeddings).reshape(1000000, D)

    mesh = plsc.VectorSubcoreMesh(core_axis_name="c", subcore_axis_name="s")
    k = pl.kernel(
        _sc_body,
        out_type=jax.ShapeDtypeStruct((total, D), jnp.float32),
        mesh=mesh,
        scratch_types=[
            pltpu.VMEM((CHUNK,), jnp.int32),
            pltpu.VMEM((CHUNK,), jnp.int32),
            pltpu.VMEM((CHUNK, D), jnp.float32),
            pltpu.VMEM((CHUNK, D), jnp.float32),
            pltpu.SemaphoreType.DMA,
            pltpu.SemaphoreType.DMA,
        ],
        compiler_params=pltpu.CompilerParams(needs_layout_passes=False,
                                             use_tc_tiling_on_sc=False),
    )
    out = k(table_rm, idx1d)
    return out.reshape(B, L, D)


# speedup vs baseline: 1.0653x; 1.0653x over previous
import jax
import jax.numpy as jnp
from jax import lax
from jax.experimental import pallas as pl
from jax.experimental.pallas import tpu as pltpu
from jax.experimental.pallas import tpu_sc as plsc

PADDING_IDX = 0
D = 32
LANES = 16
NC, NS = 2, 16
NW = NC * NS
IDX_MINOR = 128
CHUNK = 1280
VREGS = CHUNK // LANES


def _zero_pad_rows(idx_v, rows_v, k):
    iv = idx_v[pl.ds(k * LANES, LANES)]
    m = iv == PADDING_IDX

    @pl.when(jnp.any(m))
    def _():
        row_ix = jnp.arange(LANES, dtype=jnp.int32) + k * LANES
        zeros = jnp.zeros((LANES,), dtype=jnp.float32)

        def col(j, carry):
            col_ix = jnp.full((LANES,), j, dtype=jnp.int32)
            plsc.store_scatter(rows_v, [row_ix, col_ix], zeros, mask=m)
            return carry

        lax.fori_loop(0, D, col, 0)


def _mask_fixup(idx_v, rows_v):
    def acc_any(k, m):
        return m | (idx_v[pl.ds(k * LANES, LANES)] == PADDING_IDX)

    m0 = jnp.zeros((LANES,), dtype=jnp.bool_)
    mall = lax.fori_loop(0, VREGS, acc_any, m0)

    @pl.when(jnp.any(mall))
    def _():
        def per_k(k, carry):
            _zero_pad_rows(idx_v, rows_v, k)
            return carry

        lax.fori_loop(0, VREGS, per_k, 0)


def _conv_body(embt_hbm, out_hbm, src_a, src_b, dst_a, dst_b, sem_a, sem_b):
    wid = lax.axis_index("s") * NC + lax.axis_index("c")
    n_full = 7812
    nbw = 244 + jnp.where(wid < n_full - 244 * NW, 1, 0)

    iota = jnp.arange(LANES, dtype=jnp.int32)

    def transpose_block(src, dst, ncols):
        def per_g(g, carry):
            c0 = g * 8
            for k in range(8):
                c = c0 + k
                cvec = jnp.full((LANES,), 1, jnp.int32) * c
                v_lo = plsc.load_gather(src, [iota, cvec])
                v_hi = plsc.load_gather(src, [iota + LANES, cvec])
                dst[pl.ds(c * D, LANES)] = v_lo
                dst[pl.ds(c * D + LANES, LANES)] = v_hi
            return carry

        lax.fori_loop(0, ncols // 8, per_g, 0)

    def stage(i, src, sem):
        bid = wid + i * NW
        c0 = pl.multiple_of(bid * IDX_MINOR, IDX_MINOR)
        cp = pltpu.async_copy(
            embt_hbm.at[:, pl.ds(c0, IDX_MINOR)], src, sem)
        return cp

    def drain(i, src, dst, cp):
        bid = wid + i * NW
        cp.wait()
        transpose_block(src, dst, IDX_MINOR)
        pltpu.sync_copy(dst, out_hbm.at[pl.ds(bid * (IDX_MINOR * D),
                                              IDX_MINOR * D)])

    def pair_body(p, carry):
        g = p * 2
        cpa = stage(g, src_a, sem_a)
        cpb = stage(g + 1, src_b, sem_b)
        drain(g, src_a, dst_a, cpa)
        drain(g + 1, src_b, dst_b, cpb)
        return carry

    lax.fori_loop(0, nbw // 2, pair_body, 0)

    @pl.when(nbw % 2 == 1)
    def _():
        g = nbw - 1
        cpa = stage(g, src_a, sem_a)
        drain(g, src_a, dst_a, cpa)

    @pl.when(wid == NW - 1)
    def _():
        c0 = n_full * IDX_MINOR

        def scoped(src_t):
            cp = pltpu.async_copy(embt_hbm.at[:, pl.ds(c0, 64)], src_t, sem_a)
            cp.wait()
            transpose_block(src_t, dst_a, 64)
            pltpu.sync_copy(dst_a.at[pl.ds(0, 64 * D)],
                            out_hbm.at[pl.ds(c0 * D, 64 * D)])

        pl.run_scoped(scoped, pltpu.VMEM((D, 64), jnp.float32))


def _convert(embeddings):
    embt = embeddings.T
    mesh = plsc.VectorSubcoreMesh(core_axis_name="c", subcore_axis_name="s")
    k = pl.kernel(
        _conv_body,
        out_type=jax.ShapeDtypeStruct((1000000 * D,), jnp.float32),
        mesh=mesh,
        scratch_types=[
            pltpu.VMEM((D, IDX_MINOR), jnp.float32),
            pltpu.VMEM((D, IDX_MINOR), jnp.float32),
            pltpu.VMEM((IDX_MINOR * D,), jnp.float32),
            pltpu.VMEM((IDX_MINOR * D,), jnp.float32),
            pltpu.SemaphoreType.DMA,
            pltpu.SemaphoreType.DMA,
        ],
        compiler_params=pltpu.CompilerParams(needs_layout_passes=False,
                                             use_tc_tiling_on_sc=True),
    )
    return k(embt)


def _sc_body(table_hbm, idx_hbm, out_hbm,
             idx_a, idx_b, rows_a, rows_b, sem_a, sem_b):
    wid = lax.axis_index("s") * NC + lax.axis_index("c")
    per_w = idx_hbm.shape[0] // NW
    base = wid * per_w
    n_pairs = per_w // (2 * CHUNK)

    def stage(g, idx_v, rows_v, sem):
        off = base + g * CHUNK
        pltpu.sync_copy(idx_hbm.at[pl.ds(off, CHUNK)], idx_v)
        return pltpu.async_copy(table_hbm.at[idx_v], rows_v, sem)

    def drain(g, idx_v, rows_v, cp):
        cp.wait()
        _mask_fixup(idx_v, rows_v)
        pltpu.sync_copy(rows_v, out_hbm.at[pl.ds(base + g * CHUNK, CHUNK)])

    def pair_body(p, carry):
        g = p * 2
        cpa = stage(g, idx_a, rows_a, sem_a)
        cpb = stage(g + 1, idx_b, rows_b, sem_b)
        drain(g, idx_a, rows_a, cpa)
        drain(g + 1, idx_b, rows_b, cpb)
        return carry

    lax.fori_loop(0, n_pairs, pair_body, 0)


def kernel(embeddings, inputs):
    B, L = inputs.shape
    total = B * L
    idx1d = inputs.reshape(total).astype(jnp.int32)

    table_rm = _convert(embeddings).reshape(1000000, D)

    mesh = plsc.VectorSubcoreMesh(core_axis_name="c", subcore_axis_name="s")
    k = pl.kernel(
        _sc_body,
        out_type=jax.ShapeDtypeStruct((total, D), jnp.float32),
        mesh=mesh,
        scratch_types=[
            pltpu.VMEM((CHUNK,), jnp.int32),
            pltpu.VMEM((CHUNK,), jnp.int32),
            pltpu.VMEM((CHUNK, D), jnp.float32),
            pltpu.VMEM((CHUNK, D), jnp.float32),
            pltpu.SemaphoreType.DMA,
            pltpu.SemaphoreType.DMA,
        ],
        compiler_params=pltpu.CompilerParams(needs_layout_passes=False,
                                             use_tc_tiling_on_sc=False),
    )
    out = k(table_rm, idx1d)
    return out.reshape(B, L, D)


# speedup vs baseline: 1.1944x; 1.1212x over previous
import jax
import jax.numpy as jnp
from jax import lax
from jax.experimental import pallas as pl
from jax.experimental.pallas import tpu as pltpu
from jax.experimental.pallas import tpu_sc as plsc

PADDING_IDX = 0
D = 32
LANES = 16
NC, NS = 2, 16
NW = NC * NS
IDX_MINOR = 128
CHUNK = 1280
VREGS = CHUNK // LANES


def _zero_pad_rows(idx_v, rows_v, k):
    iv = idx_v[pl.ds(k * LANES, LANES)]
    m = iv == PADDING_IDX

    @pl.when(jnp.any(m))
    def _():
        row_ix = jnp.arange(LANES, dtype=jnp.int32) + k * LANES
        zeros = jnp.zeros((LANES,), dtype=jnp.float32)

        def col(j, carry):
            col_ix = jnp.full((LANES,), j, dtype=jnp.int32)
            plsc.store_scatter(rows_v, [row_ix, col_ix], zeros, mask=m)
            return carry

        lax.fori_loop(0, D, col, 0)


def _mask_fixup(idx_v, rows_v):
    def acc_any(k, m):
        return m | (idx_v[pl.ds(k * LANES, LANES)] == PADDING_IDX)

    m0 = jnp.zeros((LANES,), dtype=jnp.bool_)
    mall = lax.fori_loop(0, VREGS, acc_any, m0)

    @pl.when(jnp.any(mall))
    def _():
        def per_k(k, carry):
            _zero_pad_rows(idx_v, rows_v, k)
            return carry

        lax.fori_loop(0, VREGS, per_k, 0)


def _conv_body(embt_hbm, out_hbm, src_a, src_b, dst_a, dst_b, sem_a, sem_b):
    wid = lax.axis_index("s") * NC + lax.axis_index("c")
    n_full = 7812
    nbw = 244 + jnp.where(wid < n_full - 244 * NW, 1, 0)

    iota = jnp.arange(LANES, dtype=jnp.int32)

    def transpose_block(src, dst, ncols):
        GRP = 16

        def per_g(g, carry):
            c0 = g * GRP
            vals = []
            for k in range(GRP):
                cvec = jnp.full((LANES,), 1, jnp.int32) * (c0 + k)
                vals.append(plsc.load_gather(src, [iota, cvec]))
                vals.append(plsc.load_gather(src, [iota + LANES, cvec]))
            for k in range(GRP):
                c = c0 + k
                dst[pl.ds(c * D, LANES)] = vals[2 * k]
                dst[pl.ds(c * D + LANES, LANES)] = vals[2 * k + 1]
            return carry

        lax.fori_loop(0, ncols // GRP, per_g, 0)

    def stage(i, src, sem):
        bid = wid + i * NW
        c0 = pl.multiple_of(bid * IDX_MINOR, IDX_MINOR)
        cp = pltpu.async_copy(
            embt_hbm.at[:, pl.ds(c0, IDX_MINOR)], src, sem)
        return cp

    def drain(i, src, dst, cp):
        bid = wid + i * NW
        cp.wait()
        transpose_block(src, dst, IDX_MINOR)
        pltpu.sync_copy(dst, out_hbm.at[pl.ds(bid * (IDX_MINOR * D),
                                              IDX_MINOR * D)])

    def pair_body(p, carry):
        g = p * 2
        cpa = stage(g, src_a, sem_a)
        cpb = stage(g + 1, src_b, sem_b)
        drain(g, src_a, dst_a, cpa)
        drain(g + 1, src_b, dst_b, cpb)
        return carry

    lax.fori_loop(0, nbw // 2, pair_body, 0)

    @pl.when(nbw % 2 == 1)
    def _():
        g = nbw - 1
        cpa = stage(g, src_a, sem_a)
        drain(g, src_a, dst_a, cpa)

    @pl.when(wid == NW - 1)
    def _():
        c0 = n_full * IDX_MINOR

        def scoped(src_t):
            cp = pltpu.async_copy(embt_hbm.at[:, pl.ds(c0, 64)], src_t, sem_a)
            cp.wait()
            transpose_block(src_t, dst_a, 64)
            pltpu.sync_copy(dst_a.at[pl.ds(0, 64 * D)],
                            out_hbm.at[pl.ds(c0 * D, 64 * D)])

        pl.run_scoped(scoped, pltpu.VMEM((D, 64), jnp.float32))


def _convert(embeddings):
    embt = embeddings.T
    mesh = plsc.VectorSubcoreMesh(core_axis_name="c", subcore_axis_name="s")
    k = pl.kernel(
        _conv_body,
        out_type=jax.ShapeDtypeStruct((1000000 * D,), jnp.float32),
        mesh=mesh,
        scratch_types=[
            pltpu.VMEM((D, IDX_MINOR), jnp.float32),
            pltpu.VMEM((D, IDX_MINOR), jnp.float32),
            pltpu.VMEM((IDX_MINOR * D,), jnp.float32),
            pltpu.VMEM((IDX_MINOR * D,), jnp.float32),
            pltpu.SemaphoreType.DMA,
            pltpu.SemaphoreType.DMA,
        ],
        compiler_params=pltpu.CompilerParams(needs_layout_passes=False,
                                             use_tc_tiling_on_sc=True),
    )
    return k(embt)


def _sc_body(table_hbm, idx_hbm, out_hbm,
             idx_a, idx_b, rows_a, rows_b, sem_a, sem_b):
    wid = lax.axis_index("s") * NC + lax.axis_index("c")
    per_w = idx_hbm.shape[0] // NW
    base = wid * per_w
    n_pairs = per_w // (2 * CHUNK)

    def stage(g, idx_v, rows_v, sem):
        off = base + g * CHUNK
        pltpu.sync_copy(idx_hbm.at[pl.ds(off, CHUNK)], idx_v)
        return pltpu.async_copy(table_hbm.at[idx_v], rows_v, sem)

    def drain(g, idx_v, rows_v, cp):
        cp.wait()
        _mask_fixup(idx_v, rows_v)
        pltpu.sync_copy(rows_v, out_hbm.at[pl.ds(base + g * CHUNK, CHUNK)])

    def pair_body(p, carry):
        g = p * 2
        cpa = stage(g, idx_a, rows_a, sem_a)
        cpb = stage(g + 1, idx_b, rows_b, sem_b)
        drain(g, idx_a, rows_a, cpa)
        drain(g + 1, idx_b, rows_b, cpb)
        return carry

    lax.fori_loop(0, n_pairs, pair_body, 0)


def kernel(embeddings, inputs):
    B, L = inputs.shape
    total = B * L
    idx1d = inputs.reshape(total).astype(jnp.int32)

    table_rm = _convert(embeddings).reshape(1000000, D)

    mesh = plsc.VectorSubcoreMesh(core_axis_name="c", subcore_axis_name="s")
    k = pl.kernel(
        _sc_body,
        out_type=jax.ShapeDtypeStruct((total, D), jnp.float32),
        mesh=mesh,
        scratch_types=[
            pltpu.VMEM((CHUNK,), jnp.int32),
            pltpu.VMEM((CHUNK,), jnp.int32),
            pltpu.VMEM((CHUNK, D), jnp.float32),
            pltpu.VMEM((CHUNK, D), jnp.float32),
            pltpu.SemaphoreType.DMA,
            pltpu.SemaphoreType.DMA,
        ],
        compiler_params=pltpu.CompilerParams(needs_layout_passes=False,
                                             use_tc_tiling_on_sc=False),
    )
    out = k(table_rm, idx1d)
    return out.reshape(B, L, D)


# speedup vs baseline: 1.8453x; 1.5450x over previous
import jax
import jax.numpy as jnp
from jax import lax
from jax.experimental import pallas as pl
from jax.experimental.pallas import tpu as pltpu
from jax.experimental.pallas import tpu_sc as plsc

PADDING_IDX = 0
D = 32
LANES = 16
NC, NS = 2, 16
NW = NC * NS
IDX_MINOR = 128
CHUNK = 1280
VREGS = CHUNK // LANES


def _zero_pad_rows(idx_v, rows_v, k):
    iv = idx_v[pl.ds(k * LANES, LANES)]
    m = iv == PADDING_IDX

    @pl.when(jnp.any(m))
    def _():
        row_ix = jnp.arange(LANES, dtype=jnp.int32) + k * LANES
        zeros = jnp.zeros((LANES,), dtype=jnp.float32)

        def col(j, carry):
            col_ix = jnp.full((LANES,), j, dtype=jnp.int32)
            plsc.store_scatter(rows_v, [row_ix, col_ix], zeros, mask=m)
            return carry

        lax.fori_loop(0, D, col, 0)


def _mask_fixup(idx_v, rows_v):
    def acc_any(k, m):
        return m | (idx_v[pl.ds(k * LANES, LANES)] == PADDING_IDX)

    m0 = jnp.zeros((LANES,), dtype=jnp.bool_)
    mall = lax.fori_loop(0, VREGS, acc_any, m0)

    @pl.when(jnp.any(mall))
    def _():
        def per_k(k, carry):
            _zero_pad_rows(idx_v, rows_v, k)
            return carry

        lax.fori_loop(0, VREGS, per_k, 0)


def _conv_body(embt_hbm, out_hbm, src_a, src_b, dst_a, dst_b, sem_a, sem_b):
    wid = lax.axis_index("s") * NC + lax.axis_index("c")
    n_full = 7812
    nbw = 244 + jnp.where(wid < n_full - 244 * NW, 1, 0)

    iota = jnp.arange(LANES, dtype=jnp.int32)

    perm = [(iota + d) & (LANES - 1) for d in range(LANES)]
    sd = [p * D + iota for p in perm]

    def transpose_block(src, dst, ncols):
        def per_c0(g, carry):
            c0 = g * LANES
            for j0 in (0, LANES):
                vals = []
                for d in range(LANES):
                    vals.append(plsc.load_gather(src,
                                                 [iota + j0, perm[d] + c0]))
                for d in range(LANES):
                    plsc.store_scatter(dst, [sd[d] + (c0 * D + j0)], vals[d])
            return carry

        lax.fori_loop(0, ncols // LANES, per_c0, 0)

    def stage(i, src, sem):
        bid = wid + i * NW
        c0 = pl.multiple_of(bid * IDX_MINOR, IDX_MINOR)
        cp = pltpu.async_copy(
            embt_hbm.at[:, pl.ds(c0, IDX_MINOR)], src, sem)
        return cp

    def drain(i, src, dst, cp):
        bid = wid + i * NW
        cp.wait()
        transpose_block(src, dst, IDX_MINOR)
        pltpu.sync_copy(dst, out_hbm.at[pl.ds(bid * (IDX_MINOR * D),
                                              IDX_MINOR * D)])

    def pair_body(p, carry):
        g = p * 2
        cpa = stage(g, src_a, sem_a)
        cpb = stage(g + 1, src_b, sem_b)
        drain(g, src_a, dst_a, cpa)
        drain(g + 1, src_b, dst_b, cpb)
        return carry

    lax.fori_loop(0, nbw // 2, pair_body, 0)

    @pl.when(nbw % 2 == 1)
    def _():
        g = nbw - 1
        cpa = stage(g, src_a, sem_a)
        drain(g, src_a, dst_a, cpa)

    @pl.when(wid == NW - 1)
    def _():
        c0 = n_full * IDX_MINOR

        def scoped(src_t):
            cp = pltpu.async_copy(embt_hbm.at[:, pl.ds(c0, 64)], src_t, sem_a)
            cp.wait()
            transpose_block(src_t, dst_a, 64)
            pltpu.sync_copy(dst_a.at[pl.ds(0, 64 * D)],
                            out_hbm.at[pl.ds(c0 * D, 64 * D)])

        pl.run_scoped(scoped, pltpu.VMEM((D, 64), jnp.float32))


def _convert(embeddings):
    embt = embeddings.T
    mesh = plsc.VectorSubcoreMesh(core_axis_name="c", subcore_axis_name="s")
    k = pl.kernel(
        _conv_body,
        out_type=jax.ShapeDtypeStruct((1000000 * D,), jnp.float32),
        mesh=mesh,
        scratch_types=[
            pltpu.VMEM((D, IDX_MINOR), jnp.float32),
            pltpu.VMEM((D, IDX_MINOR), jnp.float32),
            pltpu.VMEM((IDX_MINOR * D,), jnp.float32),
            pltpu.VMEM((IDX_MINOR * D,), jnp.float32),
            pltpu.SemaphoreType.DMA,
            pltpu.SemaphoreType.DMA,
        ],
        compiler_params=pltpu.CompilerParams(needs_layout_passes=False,
                                             use_tc_tiling_on_sc=True),
    )
    return k(embt)


def _sc_body(table_hbm, idx_hbm, out_hbm,
             idx_a, idx_b, rows_a, rows_b, sem_a, sem_b):
    wid = lax.axis_index("s") * NC + lax.axis_index("c")
    per_w = idx_hbm.shape[0] // NW
    base = wid * per_w
    n_pairs = per_w // (2 * CHUNK)

    def stage(g, idx_v, rows_v, sem):
        off = base + g * CHUNK
        pltpu.sync_copy(idx_hbm.at[pl.ds(off, CHUNK)], idx_v)
        return pltpu.async_copy(table_hbm.at[idx_v], rows_v, sem)

    def drain(g, idx_v, rows_v, cp):
        cp.wait()
        _mask_fixup(idx_v, rows_v)
        pltpu.sync_copy(rows_v, out_hbm.at[pl.ds(base + g * CHUNK, CHUNK)])

    def pair_body(p, carry):
        g = p * 2
        cpa = stage(g, idx_a, rows_a, sem_a)
        cpb = stage(g + 1, idx_b, rows_b, sem_b)
        drain(g, idx_a, rows_a, cpa)
        drain(g + 1, idx_b, rows_b, cpb)
        return carry

    lax.fori_loop(0, n_pairs, pair_body, 0)


def kernel(embeddings, inputs):
    B, L = inputs.shape
    total = B * L
    idx1d = inputs.reshape(total).astype(jnp.int32)

    table_rm = _convert(embeddings).reshape(1000000, D)

    mesh = plsc.VectorSubcoreMesh(core_axis_name="c", subcore_axis_name="s")
    k = pl.kernel(
        _sc_body,
        out_type=jax.ShapeDtypeStruct((total, D), jnp.float32),
        mesh=mesh,
        scratch_types=[
            pltpu.VMEM((CHUNK,), jnp.int32),
            pltpu.VMEM((CHUNK,), jnp.int32),
            pltpu.VMEM((CHUNK, D), jnp.float32),
            pltpu.VMEM((CHUNK, D), jnp.float32),
            pltpu.SemaphoreType.DMA,
            pltpu.SemaphoreType.DMA,
        ],
        compiler_params=pltpu.CompilerParams(needs_layout_passes=False,
                                             use_tc_tiling_on_sc=False),
    )
    out = k(table_rm, idx1d)
    return out.reshape(B, L, D)


# speedup vs baseline: 3.6148x; 1.9589x over previous
import jax
import jax.numpy as jnp
from jax import lax
from jax.experimental import pallas as pl
from jax.experimental.pallas import tpu as pltpu
from jax.experimental.pallas import tpu_sc as plsc

PADDING_IDX = 0
D = 32
LANES = 16
NC, NS = 2, 16
NW = NC * NS
IDX_MINOR = 128
CHUNK = 1280
VREGS = CHUNK // LANES


def _conv_body(embt_hbm, out_hbm, src_a, src_b, dst_a, dst_b, sem_a, sem_b):
    wid = lax.axis_index("s") * NC + lax.axis_index("c")
    n_full = 7812
    nbw = 244 + jnp.where(wid < n_full - 244 * NW, 1, 0)

    iota = jnp.arange(LANES, dtype=jnp.int32)

    perm = [(iota + d) & (LANES - 1) for d in range(LANES)]
    sd = [p * D + iota for p in perm]

    def transpose_block(src, dst, ncols):
        def per_c0(g, carry):
            c0 = g * LANES
            for j0 in (0, LANES):
                vals = []
                for d in range(LANES):
                    vals.append(plsc.load_gather(src,
                                                 [iota + j0, perm[d] + c0]))
                for d in range(LANES):
                    plsc.store_scatter(dst, [sd[d] + (c0 * D + j0)], vals[d])
            return carry

        lax.fori_loop(0, ncols // LANES, per_c0, 0)

    def stage(i, src, sem):
        bid = wid + i * NW
        c0 = pl.multiple_of(bid * IDX_MINOR, IDX_MINOR)
        cp = pltpu.async_copy(
            embt_hbm.at[:, pl.ds(c0, IDX_MINOR)], src, sem)
        return cp

    def drain(i, src, dst, cp):
        bid = wid + i * NW
        cp.wait()
        transpose_block(src, dst, IDX_MINOR)
        pltpu.sync_copy(dst, out_hbm.at[pl.ds(bid * (IDX_MINOR * D),
                                              IDX_MINOR * D)])

    def pair_body(p, carry):
        g = p * 2
        cpa = stage(g, src_a, sem_a)
        cpb = stage(g + 1, src_b, sem_b)
        drain(g, src_a, dst_a, cpa)
        drain(g + 1, src_b, dst_b, cpb)
        return carry

    lax.fori_loop(0, nbw // 2, pair_body, 0)

    @pl.when(nbw % 2 == 1)
    def _():
        g = nbw - 1
        cpa = stage(g, src_a, sem_a)
        drain(g, src_a, dst_a, cpa)

    @pl.when(wid == NW - 1)
    def _():
        c0 = n_full * IDX_MINOR

        def scoped(src_t):
            cp = pltpu.async_copy(embt_hbm.at[:, pl.ds(c0, 64)], src_t, sem_a)
            cp.wait()
            transpose_block(src_t, dst_a, 64)
            pltpu.sync_copy(dst_a.at[pl.ds(0, 64 * D)],
                            out_hbm.at[pl.ds(c0 * D, 64 * D)])

        pl.run_scoped(scoped, pltpu.VMEM((D, 64), jnp.float32))


def _convert(embeddings):
    embt = embeddings.T
    mesh = plsc.VectorSubcoreMesh(core_axis_name="c", subcore_axis_name="s")
    k = pl.kernel(
        _conv_body,
        out_type=jax.ShapeDtypeStruct((1000000 * D,), jnp.float32),
        mesh=mesh,
        scratch_types=[
            pltpu.VMEM((D, IDX_MINOR), jnp.float32),
            pltpu.VMEM((D, IDX_MINOR), jnp.float32),
            pltpu.VMEM((IDX_MINOR * D,), jnp.float32),
            pltpu.VMEM((IDX_MINOR * D,), jnp.float32),
            pltpu.SemaphoreType.DMA,
            pltpu.SemaphoreType.DMA,
        ],
        compiler_params=pltpu.CompilerParams(needs_layout_passes=False,
                                             use_tc_tiling_on_sc=True),
    )
    return k(embt)


CL = 5
NLINE = CL * 4


def _sc_body(table_hbm, idxt_hbm, out_hbm,
             idx_a, idx_b, rows_a, rows_b, xp_a, xp_b,
             sem_a, sem_b, sem_oa, sem_ob):
    w = lax.axis_index("s") * NC + lax.axis_index("c")
    iota = jnp.arange(LANES, dtype=jnp.int32)
    perm = [(iota + d) & (LANES - 1) for d in range(LANES)]
    sd2 = [(p // 8) * 1024 + (p % 8) * IDX_MINOR + iota for p in perm]
    n_chunks = idxt_hbm.shape[0] // CL
    zero = jnp.zeros((LANES,), dtype=jnp.float32)

    def stage(g, idx1, rows, sem):
        l0 = g * CL
        cps = [pltpu.async_copy(
                   idxt_hbm.at[l0 + li, pl.ds(w * IDX_MINOR, IDX_MINOR)],
                   idx1.at[pl.ds(li * IDX_MINOR, IDX_MINOR)], sem)
               for li in range(CL)]
        for cp in cps:
            cp.wait()
        return pltpu.async_copy(table_hbm.at[idx1], rows, sem)

    def transpose_chunk(idx2, rows, xp):
        def per_g(gg, carry):
            l = gg // 8
            b0 = (gg % 8) * LANES
            iv = idx2[pl.ds(l * IDX_MINOR + b0, LANES)]
            mask = iv != PADDING_IDX
            rowv = l * IDX_MINOR + b0 + iota
            for j0 in (0, LANES):
                vals = [plsc.load_gather(rows, [rowv, perm[d] + j0])
                        for d in range(LANES)]
                base2 = (l * 4 + j0 // 8) * 1024 + b0
                for d in range(LANES):
                    vm = jnp.where(mask, vals[d], zero)
                    plsc.store_scatter(xp, [sd2[d] + base2], vm)
            return carry

        lax.fori_loop(0, CL * 8, per_g, 0)

    def outdma(g, xp, sem):
        l0 = g * CL
        for li in range(CL):
            for jt in range(4):
                n = ((l0 + li) * 4 + jt) * NW + w
                pltpu.async_copy(xp.at[pl.ds((li * 4 + jt) * 1024, 1024)],
                                 out_hbm.at[pl.ds(n * 1024, 1024)], sem)

    def drain_out(xp, sem):
        pltpu.make_async_copy(out_hbm.at[pl.ds(0, NLINE * 1024)], xp,
                              sem).wait()

    def pair_body(p, carry):
        g = p * 2
        cpa = stage(g, idx_a, rows_a, sem_a)
        cpb = stage(g + 1, idx_b, rows_b, sem_b)
        cpa.wait()

        @pl.when(p > 0)
        def _():
            drain_out(xp_a, sem_oa)

        transpose_chunk(idx_a, rows_a, xp_a)
        outdma(g, xp_a, sem_oa)
        cpb.wait()

        @pl.when(p > 0)
        def _():
            drain_out(xp_b, sem_ob)

        transpose_chunk(idx_b, rows_b, xp_b)
        outdma(g + 1, xp_b, sem_ob)
        return carry

    lax.fori_loop(0, n_chunks // 2, pair_body, 0)
    drain_out(xp_a, sem_oa)
    drain_out(xp_b, sem_ob)


def kernel(embeddings, inputs):
    B, L = inputs.shape

    table_rm = _convert(embeddings).reshape(1000000, D)
    idxt = inputs.T.astype(jnp.int32)

    mesh = plsc.VectorSubcoreMesh(core_axis_name="c", subcore_axis_name="s")
    k = pl.kernel(
        _sc_body,
        out_type=jax.ShapeDtypeStruct((L * 4 * NW * 8 * IDX_MINOR,),
                                      jnp.float32),
        mesh=mesh,
        scratch_types=[
            pltpu.VMEM((CL * IDX_MINOR,), jnp.int32),
            pltpu.VMEM((CL * IDX_MINOR,), jnp.int32),
            pltpu.VMEM((CL * IDX_MINOR, D), jnp.float32),
            pltpu.VMEM((CL * IDX_MINOR, D), jnp.float32),
            pltpu.VMEM((NLINE * 8 * IDX_MINOR,), jnp.float32),
            pltpu.VMEM((NLINE * 8 * IDX_MINOR,), jnp.float32),
            pltpu.SemaphoreType.DMA,
            pltpu.SemaphoreType.DMA,
            pltpu.SemaphoreType.DMA,
            pltpu.SemaphoreType.DMA,
        ],
        compiler_params=pltpu.CompilerParams(needs_layout_passes=False,
                                             use_tc_tiling_on_sc=False),
    )
    out4 = k(table_rm, idxt)
    out = (out4.reshape(L, 4, NW, 8, IDX_MINOR)
           .transpose(2, 4, 0, 1, 3)
           .reshape(B, L, D))
    return out


# speedup vs baseline: 4.2039x; 1.1630x over previous
import jax
import jax.numpy as jnp
from jax import lax
from jax.experimental import pallas as pl
from jax.experimental.pallas import tpu as pltpu
from jax.experimental.pallas import tpu_sc as plsc

PADDING_IDX = 0
D = 32
LANES = 16
NC, NS = 2, 16
NW = NC * NS
IDX_MINOR = 128
CONV_W = 256
CHUNK = 1280
VREGS = CHUNK // LANES


def _conv_body(embt_hbm, out_hbm, src_a, src_b, dst_a, dst_b, sem_a, sem_b):
    wid = lax.axis_index("s") * NC + lax.axis_index("c")
    n_full = 1000000 // CONV_W
    base_n = n_full // NW
    nbw = base_n + jnp.where(wid < n_full - base_n * NW, 1, 0)

    iota = jnp.arange(LANES, dtype=jnp.int32)

    perm = [(iota + d) & (LANES - 1) for d in range(LANES)]
    sd = [p * D + iota for p in perm]

    def transpose_block(src, dst, ncols):
        def per_c0(g, carry):
            c0 = g * LANES
            for j0 in (0, LANES):
                vals = []
                for d in range(LANES):
                    vals.append(plsc.load_gather(src,
                                                 [iota + j0, perm[d] + c0]))
                for d in range(LANES):
                    plsc.store_scatter(dst, [sd[d] + (c0 * D + j0)], vals[d])
            return carry

        lax.fori_loop(0, ncols // LANES, per_c0, 0)

    def stage(i, src, sem):
        bid = wid + i * NW
        c0 = pl.multiple_of(bid * CONV_W, IDX_MINOR)
        cp = pltpu.async_copy(
            embt_hbm.at[:, pl.ds(c0, CONV_W)], src, sem)
        return cp

    def drain(i, src, dst, cp):
        bid = wid + i * NW
        cp.wait()
        transpose_block(src, dst, CONV_W)
        pltpu.sync_copy(dst, out_hbm.at[pl.ds(bid * (CONV_W * D),
                                              CONV_W * D)])

    def pair_body(p, carry):
        g = p * 2
        cpa = stage(g, src_a, sem_a)
        cpb = stage(g + 1, src_b, sem_b)
        drain(g, src_a, dst_a, cpa)
        drain(g + 1, src_b, dst_b, cpb)
        return carry

    lax.fori_loop(0, nbw // 2, pair_body, 0)

    @pl.when(nbw % 2 == 1)
    def _():
        g = nbw - 1
        cpa = stage(g, src_a, sem_a)
        drain(g, src_a, dst_a, cpa)

    @pl.when(wid == NW - 1)
    def _():
        c0 = n_full * CONV_W

        def scoped(src_t):
            cp = pltpu.async_copy(embt_hbm.at[:, pl.ds(c0, 64)], src_t, sem_a)
            cp.wait()
            transpose_block(src_t, dst_a, 64)
            pltpu.sync_copy(dst_a.at[pl.ds(0, 64 * D)],
                            out_hbm.at[pl.ds(c0 * D, 64 * D)])

        pl.run_scoped(scoped, pltpu.VMEM((D, 64), jnp.float32))


def _convert(embeddings):
    embt = embeddings.T
    mesh = plsc.VectorSubcoreMesh(core_axis_name="c", subcore_axis_name="s")
    k = pl.kernel(
        _conv_body,
        out_type=jax.ShapeDtypeStruct((1000000 * D,), jnp.float32),
        mesh=mesh,
        scratch_types=[
            pltpu.VMEM((D, CONV_W), jnp.float32),
            pltpu.VMEM((D, CONV_W), jnp.float32),
            pltpu.VMEM((CONV_W * D,), jnp.float32),
            pltpu.VMEM((CONV_W * D,), jnp.float32),
            pltpu.SemaphoreType.DMA,
            pltpu.SemaphoreType.DMA,
        ],
        compiler_params=pltpu.CompilerParams(needs_layout_passes=False,
                                             use_tc_tiling_on_sc=True),
    )
    return k(embt)


CL = 5
NLINE = CL * 4


def _sc_body(table_hbm, idxt_hbm, out_hbm,
             idx_a, idx_b, rows_a, rows_b, xp_a, xp_b,
             sem_a, sem_b, sem_oa, sem_ob):
    w = lax.axis_index("s") * NC + lax.axis_index("c")
    iota = jnp.arange(LANES, dtype=jnp.int32)
    perm = [(iota + d) & (LANES - 1) for d in range(LANES)]
    sd2 = [(p // 8) * 1024 + (p % 8) * IDX_MINOR + iota for p in perm]
    n_chunks = idxt_hbm.shape[0] // CL
    zero = jnp.zeros((LANES,), dtype=jnp.float32)

    def stage(g, idx1, rows, sem):
        l0 = g * CL
        cps = [pltpu.async_copy(
                   idxt_hbm.at[l0 + li, pl.ds(w * IDX_MINOR, IDX_MINOR)],
                   idx1.at[pl.ds(li * IDX_MINOR, IDX_MINOR)], sem)
               for li in range(CL)]
        for cp in cps:
            cp.wait()
        return pltpu.async_copy(table_hbm.at[idx1], rows, sem)

    def transpose_chunk(idx2, rows, xp):
        def per_g(gg, carry):
            l = gg // 8
            b0 = (gg % 8) * LANES
            iv = idx2[pl.ds(l * IDX_MINOR + b0, LANES)]
            mask = iv != PADDING_IDX
            rowv = l * IDX_MINOR + b0 + iota
            for j0 in (0, LANES):
                vals = [plsc.load_gather(rows, [rowv, perm[d] + j0])
                        for d in range(LANES)]
                base2 = (l * 4 + j0 // 8) * 1024 + b0
                for d in range(LANES):
                    vm = jnp.where(mask, vals[d], zero)
                    plsc.store_scatter(xp, [sd2[d] + base2], vm)
            return carry

        lax.fori_loop(0, CL * 8, per_g, 0)

    def outdma(g, xp, sem):
        l0 = g * CL
        for li in range(CL):
            for jt in range(4):
                n = ((l0 + li) * 4 + jt) * NW + w
                pltpu.async_copy(xp.at[pl.ds((li * 4 + jt) * 1024, 1024)],
                                 out_hbm.at[pl.ds(n * 1024, 1024)], sem)

    def drain_out(xp, sem):
        pltpu.make_async_copy(out_hbm.at[pl.ds(0, NLINE * 1024)], xp,
                              sem).wait()

    def pair_body(p, carry):
        g = p * 2
        cpa = stage(g, idx_a, rows_a, sem_a)
        cpb = stage(g + 1, idx_b, rows_b, sem_b)
        cpa.wait()

        @pl.when(p > 0)
        def _():
            drain_out(xp_a, sem_oa)

        transpose_chunk(idx_a, rows_a, xp_a)
        outdma(g, xp_a, sem_oa)
        cpb.wait()

        @pl.when(p > 0)
        def _():
            drain_out(xp_b, sem_ob)

        transpose_chunk(idx_b, rows_b, xp_b)
        outdma(g + 1, xp_b, sem_ob)
        return carry

    lax.fori_loop(0, n_chunks // 2, pair_body, 0)
    drain_out(xp_a, sem_oa)
    drain_out(xp_b, sem_ob)


def kernel(embeddings, inputs):
    B, L = inputs.shape

    table_rm = _convert(embeddings).reshape(1000000, D)
    idxt = inputs.T.astype(jnp.int32)

    mesh = plsc.VectorSubcoreMesh(core_axis_name="c", subcore_axis_name="s")
    k = pl.kernel(
        _sc_body,
        out_type=jax.ShapeDtypeStruct((L * 4 * NW * 8 * IDX_MINOR,),
                                      jnp.float32),
        mesh=mesh,
        scratch_types=[
            pltpu.VMEM((CL * IDX_MINOR,), jnp.int32),
            pltpu.VMEM((CL * IDX_MINOR,), jnp.int32),
            pltpu.VMEM((CL * IDX_MINOR, D), jnp.float32),
            pltpu.VMEM((CL * IDX_MINOR, D), jnp.float32),
            pltpu.VMEM((NLINE * 8 * IDX_MINOR,), jnp.float32),
            pltpu.VMEM((NLINE * 8 * IDX_MINOR,), jnp.float32),
            pltpu.SemaphoreType.DMA,
            pltpu.SemaphoreType.DMA,
            pltpu.SemaphoreType.DMA,
            pltpu.SemaphoreType.DMA,
        ],
        compiler_params=pltpu.CompilerParams(needs_layout_passes=False,
                                             use_tc_tiling_on_sc=False),
    )
    out4 = k(table_rm, idxt)
    out = (out4.reshape(L, 4, NW, 8, IDX_MINOR)
           .transpose(2, 4, 0, 1, 3)
           .reshape(B, L, D))
    return out


# speedup vs baseline: 4.5168x; 1.0744x over previous
import jax
import jax.numpy as jnp
from jax import lax
from jax.experimental import pallas as pl
from jax.experimental.pallas import tpu as pltpu
from jax.experimental.pallas import tpu_sc as plsc

PADDING_IDX = 0
D = 32
LANES = 16
NC, NS = 2, 16
NW = NC * NS
IDX_MINOR = 128
CONV_W = 512
CHUNK = 1280
VREGS = CHUNK // LANES


def _conv_body(embt_hbm, out_hbm, src_a, src_b, dst_a, dst_b, sem_a, sem_b):
    wid = lax.axis_index("s") * NC + lax.axis_index("c")
    n_full = 1000000 // CONV_W
    base_n = n_full // NW
    nbw = base_n + jnp.where(wid < n_full - base_n * NW, 1, 0)

    iota = jnp.arange(LANES, dtype=jnp.int32)

    perm = [(iota + d) & (LANES - 1) for d in range(LANES)]
    sd = [p * D + iota for p in perm]

    def transpose_block(src, dst, ncols):
        def per_c0(g, carry):
            c0 = g * LANES
            for j0 in (0, LANES):
                vals = []
                for d in range(LANES):
                    vals.append(plsc.load_gather(src,
                                                 [iota + j0, perm[d] + c0]))
                for d in range(LANES):
                    plsc.store_scatter(dst, [sd[d] + (c0 * D + j0)], vals[d])
            return carry

        lax.fori_loop(0, ncols // LANES, per_c0, 0)

    def stage(i, src, sem):
        bid = wid + i * NW
        c0 = pl.multiple_of(bid * CONV_W, IDX_MINOR)
        cp = pltpu.async_copy(
            embt_hbm.at[:, pl.ds(c0, CONV_W)], src, sem)
        return cp

    def drain(i, src, dst, cp):
        bid = wid + i * NW
        cp.wait()
        transpose_block(src, dst, CONV_W)
        pltpu.sync_copy(dst, out_hbm.at[pl.ds(bid * (CONV_W * D),
                                              CONV_W * D)])

    def pair_body(p, carry):
        g = p * 2
        cpa = stage(g, src_a, sem_a)
        cpb = stage(g + 1, src_b, sem_b)
        drain(g, src_a, dst_a, cpa)
        drain(g + 1, src_b, dst_b, cpb)
        return carry

    lax.fori_loop(0, nbw // 2, pair_body, 0)

    @pl.when(nbw % 2 == 1)
    def _():
        g = nbw - 1
        cpa = stage(g, src_a, sem_a)
        drain(g, src_a, dst_a, cpa)

    @pl.when(wid == NW - 1)
    def _():
        c0 = n_full * CONV_W

        def scoped(src_t):
            cp = pltpu.async_copy(embt_hbm.at[:, pl.ds(c0, 64)], src_t, sem_a)
            cp.wait()
            transpose_block(src_t, dst_a, 64)
            pltpu.sync_copy(dst_a.at[pl.ds(0, 64 * D)],
                            out_hbm.at[pl.ds(c0 * D, 64 * D)])

        pl.run_scoped(scoped, pltpu.VMEM((D, 64), jnp.float32))


def _convert(embeddings):
    embt = embeddings.T
    mesh = plsc.VectorSubcoreMesh(core_axis_name="c", subcore_axis_name="s")
    k = pl.kernel(
        _conv_body,
        out_type=jax.ShapeDtypeStruct((1000000 * D,), jnp.float32),
        mesh=mesh,
        scratch_types=[
            pltpu.VMEM((D, CONV_W), jnp.float32),
            pltpu.VMEM((D, CONV_W), jnp.float32),
            pltpu.VMEM((CONV_W * D,), jnp.float32),
            pltpu.VMEM((CONV_W * D,), jnp.float32),
            pltpu.SemaphoreType.DMA,
            pltpu.SemaphoreType.DMA,
        ],
        compiler_params=pltpu.CompilerParams(needs_layout_passes=False,
                                             use_tc_tiling_on_sc=True),
    )
    return k(embt)


CL = 5
NLINE = CL * 4


def _sc_body(table_hbm, idxt_hbm, out_hbm,
             idx_a, idx_b, rows_a, rows_b, xp_a, xp_b,
             sem_a, sem_b, sem_oa, sem_ob):
    w = lax.axis_index("s") * NC + lax.axis_index("c")
    iota = jnp.arange(LANES, dtype=jnp.int32)
    perm = [(iota + d) & (LANES - 1) for d in range(LANES)]
    sd2 = [(p // 8) * 1024 + (p % 8) * IDX_MINOR + iota for p in perm]
    n_chunks = idxt_hbm.shape[0] // CL
    zero = jnp.zeros((LANES,), dtype=jnp.float32)

    def stage(g, idx1, rows, sem):
        l0 = g * CL
        cps = [pltpu.async_copy(
                   idxt_hbm.at[l0 + li, pl.ds(w * IDX_MINOR, IDX_MINOR)],
                   idx1.at[pl.ds(li * IDX_MINOR, IDX_MINOR)], sem)
               for li in range(CL)]
        for cp in cps:
            cp.wait()
        return pltpu.async_copy(table_hbm.at[idx1], rows, sem)

    def transpose_chunk(idx2, rows, xp):
        def per_g(gg, carry):
            l = gg // 8
            b0 = (gg % 8) * LANES
            iv = idx2[pl.ds(l * IDX_MINOR + b0, LANES)]
            mask = iv != PADDING_IDX
            rowv = l * IDX_MINOR + b0 + iota
            for j0 in (0, LANES):
                vals = [plsc.load_gather(rows, [rowv, perm[d] + j0])
                        for d in range(LANES)]
                base2 = (l * 4 + j0 // 8) * 1024 + b0
                for d in range(LANES):
                    vm = jnp.where(mask, vals[d], zero)
                    plsc.store_scatter(xp, [sd2[d] + base2], vm)
            return carry

        lax.fori_loop(0, CL * 8, per_g, 0)

    def outdma(g, xp, sem):
        l0 = g * CL
        for li in range(CL):
            for jt in range(4):
                n = ((l0 + li) * 4 + jt) * NW + w
                pltpu.async_copy(xp.at[pl.ds((li * 4 + jt) * 1024, 1024)],
                                 out_hbm.at[pl.ds(n * 1024, 1024)], sem)

    def drain_out(xp, sem):
        pltpu.make_async_copy(out_hbm.at[pl.ds(0, NLINE * 1024)], xp,
                              sem).wait()

    def pair_body(p, carry):
        g = p * 2
        cpa = stage(g, idx_a, rows_a, sem_a)
        cpb = stage(g + 1, idx_b, rows_b, sem_b)
        cpa.wait()

        @pl.when(p > 0)
        def _():
            drain_out(xp_a, sem_oa)

        transpose_chunk(idx_a, rows_a, xp_a)
        outdma(g, xp_a, sem_oa)
        cpb.wait()

        @pl.when(p > 0)
        def _():
            drain_out(xp_b, sem_ob)

        transpose_chunk(idx_b, rows_b, xp_b)
        outdma(g + 1, xp_b, sem_ob)
        return carry

    lax.fori_loop(0, n_chunks // 2, pair_body, 0)
    drain_out(xp_a, sem_oa)
    drain_out(xp_b, sem_ob)


def kernel(embeddings, inputs):
    B, L = inputs.shape

    table_rm = _convert(embeddings).reshape(1000000, D)
    idxt = inputs.T.astype(jnp.int32)

    mesh = plsc.VectorSubcoreMesh(core_axis_name="c", subcore_axis_name="s")
    k = pl.kernel(
        _sc_body,
        out_type=jax.ShapeDtypeStruct((L * 4 * NW * 8 * IDX_MINOR,),
                                      jnp.float32),
        mesh=mesh,
        scratch_types=[
            pltpu.VMEM((CL * IDX_MINOR,), jnp.int32),
            pltpu.VMEM((CL * IDX_MINOR,), jnp.int32),
            pltpu.VMEM((CL * IDX_MINOR, D), jnp.float32),
            pltpu.VMEM((CL * IDX_MINOR, D), jnp.float32),
            pltpu.VMEM((NLINE * 8 * IDX_MINOR,), jnp.float32),
            pltpu.VMEM((NLINE * 8 * IDX_MINOR,), jnp.float32),
            pltpu.SemaphoreType.DMA,
            pltpu.SemaphoreType.DMA,
            pltpu.SemaphoreType.DMA,
            pltpu.SemaphoreType.DMA,
        ],
        compiler_params=pltpu.CompilerParams(needs_layout_passes=False,
                                             use_tc_tiling_on_sc=False),
    )
    out4 = k(table_rm, idxt)
    out = (out4.reshape(L, 4, NW, 8, IDX_MINOR)
           .transpose(2, 4, 0, 1, 3)
           .reshape(B, L, D))
    return out


# speedup vs baseline: 5.0439x; 1.1167x over previous
import jax
import jax.numpy as jnp
from jax import lax
from jax.experimental import pallas as pl
from jax.experimental.pallas import tpu as pltpu
from jax.experimental.pallas import tpu_sc as plsc

PADDING_IDX = 0
D = 32
LANES = 16
NC, NS = 2, 16
NW = NC * NS
IDX_MINOR = 128
CONV_W = 512
CHUNK = 1280
VREGS = CHUNK // LANES


def _conv_body(embt_hbm, out_hbm, src_a, src_b, dst_a, dst_b,
               sem_a, sem_b, sem_da, sem_db):
    wid = lax.axis_index("s") * NC + lax.axis_index("c")
    n_full = 1000000 // CONV_W
    base_n = n_full // NW
    nbw = base_n + jnp.where(wid < n_full - base_n * NW, 1, 0)

    iota = jnp.arange(LANES, dtype=jnp.int32)

    perm = [(iota + d) & (LANES - 1) for d in range(LANES)]
    sd = [p * D + iota for p in perm]

    def transpose_block(src, dst, ncols):
        def per_c0(g, carry):
            c0 = g * LANES
            for j0 in (0, LANES):
                vals = []
                for d in range(LANES):
                    vals.append(plsc.load_gather(src,
                                                 [iota + j0, perm[d] + c0]))
                for d in range(LANES):
                    plsc.store_scatter(dst, [sd[d] + (c0 * D + j0)], vals[d])
            return carry

        lax.fori_loop(0, ncols // LANES, per_c0, 0)

    def stage(i, src, sem):
        bid = wid + i * NW
        c0 = pl.multiple_of(bid * CONV_W, IDX_MINOR)
        cp = pltpu.async_copy(
            embt_hbm.at[:, pl.ds(c0, CONV_W)], src, sem)
        return cp

    def drain_dst(dst, sem):
        pltpu.make_async_copy(out_hbm.at[pl.ds(0, CONV_W * D)], dst,
                              sem).wait()

    def drain(i, src, dst, cp, sem_d):
        bid = wid + i * NW
        cp.wait()
        transpose_block(src, dst, CONV_W)
        pltpu.async_copy(dst, out_hbm.at[pl.ds(bid * (CONV_W * D),
                                               CONV_W * D)], sem_d)

    def pair_body(p, carry):
        g = p * 2
        cpa = stage(g, src_a, sem_a)
        cpb = stage(g + 1, src_b, sem_b)
        cpa.wait()

        @pl.when(p > 0)
        def _():
            drain_dst(dst_a, sem_da)

        transpose_block(src_a, dst_a, CONV_W)
        pltpu.async_copy(dst_a, out_hbm.at[pl.ds((wid + g * NW) *
                                                 (CONV_W * D),
                                                 CONV_W * D)], sem_da)
        cpb.wait()

        @pl.when(p > 0)
        def _():
            drain_dst(dst_b, sem_db)

        transpose_block(src_b, dst_b, CONV_W)
        pltpu.async_copy(dst_b, out_hbm.at[pl.ds((wid + (g + 1) * NW) *
                                                 (CONV_W * D),
                                                 CONV_W * D)], sem_db)
        return carry

    lax.fori_loop(0, nbw // 2, pair_body, 0)

    @pl.when(nbw % 2 == 1)
    def _():
        g = nbw - 1
        cpa = stage(g, src_a, sem_a)
        cpa.wait()
        drain_dst(dst_a, sem_da)
        transpose_block(src_a, dst_a, CONV_W)
        pltpu.async_copy(dst_a, out_hbm.at[pl.ds((wid + g * NW) *
                                                 (CONV_W * D),
                                                 CONV_W * D)], sem_da)

    @pl.when(wid == NW - 1)
    def _():
        c0 = n_full * CONV_W

        def scoped(src_t):
            cp = pltpu.async_copy(embt_hbm.at[:, pl.ds(c0, 64)], src_t, sem_a)
            cp.wait()
            drain_dst(dst_b, sem_db)
            transpose_block(src_t, dst_b, 64)
            pltpu.sync_copy(dst_b.at[pl.ds(0, 64 * D)],
                            out_hbm.at[pl.ds(c0 * D, 64 * D)])

        pl.run_scoped(scoped, pltpu.VMEM((D, 64), jnp.float32))

    drain_dst(dst_a, sem_da)

    @pl.when(wid != NW - 1)
    def _():
        drain_dst(dst_b, sem_db)


def _convert(embeddings):
    embt = embeddings.T
    mesh = plsc.VectorSubcoreMesh(core_axis_name="c", subcore_axis_name="s")
    k = pl.kernel(
        _conv_body,
        out_type=jax.ShapeDtypeStruct((1000000 * D,), jnp.float32),
        mesh=mesh,
        scratch_types=[
            pltpu.VMEM((D, CONV_W), jnp.float32),
            pltpu.VMEM((D, CONV_W), jnp.float32),
            pltpu.VMEM((CONV_W * D,), jnp.float32),
            pltpu.VMEM((CONV_W * D,), jnp.float32),
            pltpu.SemaphoreType.DMA,
            pltpu.SemaphoreType.DMA,
            pltpu.SemaphoreType.DMA,
            pltpu.SemaphoreType.DMA,
        ],
        compiler_params=pltpu.CompilerParams(needs_layout_passes=False,
                                             use_tc_tiling_on_sc=True),
    )
    return k(embt)


CL = 5
NLINE = CL * 4


def _sc_body(table_hbm, idxt_hbm, out_hbm,
             idx_a, idx_b, rows_a, rows_b, xp_a, xp_b,
             sem_a, sem_b, sem_oa, sem_ob):
    w = lax.axis_index("s") * NC + lax.axis_index("c")
    iota = jnp.arange(LANES, dtype=jnp.int32)
    perm = [(iota + d) & (LANES - 1) for d in range(LANES)]
    sd2 = [(p // 8) * 1024 + (p % 8) * IDX_MINOR + iota for p in perm]
    n_chunks = idxt_hbm.shape[0] // CL
    zero = jnp.zeros((LANES,), dtype=jnp.float32)

    def stage(g, idx1, rows, sem):
        l0 = g * CL
        cps = [pltpu.async_copy(
                   idxt_hbm.at[l0 + li, pl.ds(w * IDX_MINOR, IDX_MINOR)],
                   idx1.at[pl.ds(li * IDX_MINOR, IDX_MINOR)], sem)
               for li in range(CL)]
        for cp in cps:
            cp.wait()
        return pltpu.async_copy(table_hbm.at[idx1], rows, sem)

    def transpose_chunk(idx2, rows, xp):
        def per_g(gg, carry):
            l = gg // 8
            b0 = (gg % 8) * LANES
            iv = idx2[pl.ds(l * IDX_MINOR + b0, LANES)]
            mask = iv != PADDING_IDX
            rowv = l * IDX_MINOR + b0 + iota
            for j0 in (0, LANES):
                vals = [plsc.load_gather(rows, [rowv, perm[d] + j0])
                        for d in range(LANES)]
                base2 = (l * 4 + j0 // 8) * 1024 + b0
                for d in range(LANES):
                    vm = jnp.where(mask, vals[d], zero)
                    plsc.store_scatter(xp, [sd2[d] + base2], vm)
            return carry

        lax.fori_loop(0, CL * 8, per_g, 0)

    def outdma(g, xp, sem):
        l0 = g * CL
        for li in range(CL):
            for jt in range(4):
                n = ((l0 + li) * 4 + jt) * NW + w
                pltpu.async_copy(xp.at[pl.ds((li * 4 + jt) * 1024, 1024)],
                                 out_hbm.at[pl.ds(n * 1024, 1024)], sem)

    def drain_out(xp, sem):
        pltpu.make_async_copy(out_hbm.at[pl.ds(0, NLINE * 1024)], xp,
                              sem).wait()

    def pair_body(p, carry):
        g = p * 2
        cpa = stage(g, idx_a, rows_a, sem_a)
        cpb = stage(g + 1, idx_b, rows_b, sem_b)
        cpa.wait()

        @pl.when(p > 0)
        def _():
            drain_out(xp_a, sem_oa)

        transpose_chunk(idx_a, rows_a, xp_a)
        outdma(g, xp_a, sem_oa)
        cpb.wait()

        @pl.when(p > 0)
        def _():
            drain_out(xp_b, sem_ob)

        transpose_chunk(idx_b, rows_b, xp_b)
        outdma(g + 1, xp_b, sem_ob)
        return carry

    lax.fori_loop(0, n_chunks // 2, pair_body, 0)
    drain_out(xp_a, sem_oa)
    drain_out(xp_b, sem_ob)


def kernel(embeddings, inputs):
    B, L = inputs.shape

    table_rm = _convert(embeddings).reshape(1000000, D)
    idxt = inputs.T.astype(jnp.int32)

    mesh = plsc.VectorSubcoreMesh(core_axis_name="c", subcore_axis_name="s")
    k = pl.kernel(
        _sc_body,
        out_type=jax.ShapeDtypeStruct((L * 4 * NW * 8 * IDX_MINOR,),
                                      jnp.float32),
        mesh=mesh,
        scratch_types=[
            pltpu.VMEM((CL * IDX_MINOR,), jnp.int32),
            pltpu.VMEM((CL * IDX_MINOR,), jnp.int32),
            pltpu.VMEM((CL * IDX_MINOR, D), jnp.float32),
            pltpu.VMEM((CL * IDX_MINOR, D), jnp.float32),
            pltpu.VMEM((NLINE * 8 * IDX_MINOR,), jnp.float32),
            pltpu.VMEM((NLINE * 8 * IDX_MINOR,), jnp.float32),
            pltpu.SemaphoreType.DMA,
            pltpu.SemaphoreType.DMA,
            pltpu.SemaphoreType.DMA,
            pltpu.SemaphoreType.DMA,
        ],
        compiler_params=pltpu.CompilerParams(needs_layout_passes=False,
                                             use_tc_tiling_on_sc=False),
    )
    out4 = k(table_rm, idxt)
    out = (out4.reshape(L, 4, NW, 8, IDX_MINOR)
           .transpose(2, 4, 0, 1, 3)
           .reshape(B, L, D))
    return out


# speedup vs baseline: 5.0447x; 1.0002x over previous
import jax
import jax.numpy as jnp
from jax import lax
from jax.experimental import pallas as pl
from jax.experimental.pallas import tpu as pltpu
from jax.experimental.pallas import tpu_sc as plsc

PADDING_IDX = 0
D = 32
LANES = 16
NC, NS = 2, 16
NW = NC * NS
IDX_MINOR = 128
CONV_W = 512


def _conv_body(embt_hbm, out_hbm, src_a, src_b, dst_a, dst_b,
               sem_a, sem_b, sem_da, sem_db):
    wid = lax.axis_index("s") * NC + lax.axis_index("c")
    n_full = 1000000 // CONV_W
    base_n = n_full // NW
    nbw = base_n + jnp.where(wid < n_full - base_n * NW, 1, 0)

    iota = jnp.arange(LANES, dtype=jnp.int32)

    perm = [(iota + d) & (LANES - 1) for d in range(LANES)]
    sd = [p * D + iota for p in perm]

    def transpose_block(src, dst, ncols):
        def per_c0(g, carry):
            c0 = g * LANES
            for j0 in (0, LANES):
                vals = []
                for d in range(LANES):
                    vals.append(plsc.load_gather(src,
                                                 [iota + j0, perm[d] + c0]))
                for d in range(LANES):
                    plsc.store_scatter(dst, [sd[d] + (c0 * D + j0)], vals[d])
            return carry

        lax.fori_loop(0, ncols // LANES, per_c0, 0)

    def stage(i, src, sem):
        bid = wid + i * NW
        c0 = pl.multiple_of(bid * CONV_W, IDX_MINOR)
        cp = pltpu.async_copy(
            embt_hbm.at[:, pl.ds(c0, CONV_W)], src, sem)
        return cp

    def drain_dst(dst, sem):
        pltpu.make_async_copy(out_hbm.at[pl.ds(0, CONV_W * D)], dst,
                              sem).wait()

    def drain(i, src, dst, cp, sem_d):
        bid = wid + i * NW
        cp.wait()
        transpose_block(src, dst, CONV_W)
        pltpu.async_copy(dst, out_hbm.at[pl.ds(bid * (CONV_W * D),
                                               CONV_W * D)], sem_d)

    def pair_body(p, carry):
        g = p * 2
        cpa = stage(g, src_a, sem_a)
        cpb = stage(g + 1, src_b, sem_b)
        cpa.wait()

        @pl.when(p > 0)
        def _():
            drain_dst(dst_a, sem_da)

        transpose_block(src_a, dst_a, CONV_W)
        pltpu.async_copy(dst_a, out_hbm.at[pl.ds((wid + g * NW) *
                                                 (CONV_W * D),
                                                 CONV_W * D)], sem_da)
        cpb.wait()

        @pl.when(p > 0)
        def _():
            drain_dst(dst_b, sem_db)

        transpose_block(src_b, dst_b, CONV_W)
        pltpu.async_copy(dst_b, out_hbm.at[pl.ds((wid + (g + 1) * NW) *
                                                 (CONV_W * D),
                                                 CONV_W * D)], sem_db)
        return carry

    lax.fori_loop(0, nbw // 2, pair_body, 0)

    @pl.when(nbw % 2 == 1)
    def _():
        g = nbw - 1
        cpa = stage(g, src_a, sem_a)
        cpa.wait()
        drain_dst(dst_a, sem_da)
        transpose_block(src_a, dst_a, CONV_W)
        pltpu.async_copy(dst_a, out_hbm.at[pl.ds((wid + g * NW) *
                                                 (CONV_W * D),
                                                 CONV_W * D)], sem_da)

    @pl.when(wid == NW - 1)
    def _():
        c0 = n_full * CONV_W

        def scoped(src_t):
            cp = pltpu.async_copy(embt_hbm.at[:, pl.ds(c0, 64)], src_t, sem_a)
            cp.wait()
            drain_dst(dst_b, sem_db)
            transpose_block(src_t, dst_b, 64)
            pltpu.sync_copy(dst_b.at[pl.ds(0, 64 * D)],
                            out_hbm.at[pl.ds(c0 * D, 64 * D)])

        pl.run_scoped(scoped, pltpu.VMEM((D, 64), jnp.float32))

    drain_dst(dst_a, sem_da)

    @pl.when(wid != NW - 1)
    def _():
        drain_dst(dst_b, sem_db)


def _convert(embeddings):
    embt = embeddings.T
    mesh = plsc.VectorSubcoreMesh(core_axis_name="c", subcore_axis_name="s")
    k = pl.kernel(
        _conv_body,
        out_type=jax.ShapeDtypeStruct((1000000 * D,), jnp.float32),
        mesh=mesh,
        scratch_types=[
            pltpu.VMEM((D, CONV_W), jnp.float32),
            pltpu.VMEM((D, CONV_W), jnp.float32),
            pltpu.VMEM((CONV_W * D,), jnp.float32),
            pltpu.VMEM((CONV_W * D,), jnp.float32),
            pltpu.SemaphoreType.DMA,
            pltpu.SemaphoreType.DMA,
            pltpu.SemaphoreType.DMA,
            pltpu.SemaphoreType.DMA,
        ],
        compiler_params=pltpu.CompilerParams(needs_layout_passes=False,
                                             use_tc_tiling_on_sc=True),
    )
    return k(embt)


CL = 5
NLINE = CL * 4


def _sc_body(table_hbm, idxt_hbm, out_hbm,
             idx_a, idx_b, rows_a, rows_b, xp_a, xp_b,
             sem_a, sem_b, sem_oa, sem_ob):
    w = lax.axis_index("s") * NC + lax.axis_index("c")
    iota = jnp.arange(LANES, dtype=jnp.int32)
    perm = [(iota + d) & (LANES - 1) for d in range(LANES)]
    sd2 = [(p // 8) * 1024 + (p % 8) * IDX_MINOR + iota for p in perm]
    n_chunks = idxt_hbm.shape[0] // CL
    zero = jnp.zeros((LANES,), dtype=jnp.float32)

    def stage(g, idx1, rows, sem):
        l0 = g * CL
        cps = [pltpu.async_copy(
                   idxt_hbm.at[l0 + li, pl.ds(w * IDX_MINOR, IDX_MINOR)],
                   idx1.at[pl.ds(li * IDX_MINOR, IDX_MINOR)], sem)
               for li in range(CL)]
        for cp in cps:
            cp.wait()
        return pltpu.async_copy(table_hbm.at[idx1], rows, sem)

    def transpose_chunk(idx2, rows, xp):
        def per_g(gg, carry):
            l = gg // 8
            b0 = (gg % 8) * LANES
            iv = idx2[pl.ds(l * IDX_MINOR + b0, LANES)]
            mask = iv != PADDING_IDX
            rowv = l * IDX_MINOR + b0 + iota
            for j0 in (0, LANES):
                vals = [plsc.load_gather(rows, [rowv, perm[d] + j0])
                        for d in range(LANES)]
                base2 = (l * 4 + j0 // 8) * 1024 + b0
                for d in range(LANES):
                    vm = jnp.where(mask, vals[d], zero)
                    plsc.store_scatter(xp, [sd2[d] + base2], vm)
            return carry

        lax.fori_loop(0, CL * 8, per_g, 0)

    def outdma(g, xp, sem):
        l0 = g * CL
        for li in range(CL):
            for jt in range(4):
                n = ((l0 + li) * 4 + jt) * NW + w
                pltpu.async_copy(xp.at[pl.ds((li * 4 + jt) * 1024, 1024)],
                                 out_hbm.at[pl.ds(n * 1024, 1024)], sem)

    def drain_out(xp, sem):
        pltpu.make_async_copy(out_hbm.at[pl.ds(0, NLINE * 1024)], xp,
                              sem).wait()

    def pair_body(p, carry):
        g = p * 2
        cpa = stage(g, idx_a, rows_a, sem_a)
        cpb = stage(g + 1, idx_b, rows_b, sem_b)
        cpa.wait()

        @pl.when(p > 0)
        def _():
            drain_out(xp_a, sem_oa)

        transpose_chunk(idx_a, rows_a, xp_a)
        outdma(g, xp_a, sem_oa)
        cpb.wait()

        @pl.when(p > 0)
        def _():
            drain_out(xp_b, sem_ob)

        transpose_chunk(idx_b, rows_b, xp_b)
        outdma(g + 1, xp_b, sem_ob)
        return carry

    lax.fori_loop(0, n_chunks // 2, pair_body, 0)
    drain_out(xp_a, sem_oa)
    drain_out(xp_b, sem_ob)


def kernel(embeddings, inputs):
    B, L = inputs.shape

    table_rm = _convert(embeddings).reshape(1000000, D)
    idxt = inputs.T.astype(jnp.int32)

    mesh = plsc.VectorSubcoreMesh(core_axis_name="c", subcore_axis_name="s")
    k = pl.kernel(
        _sc_body,
        out_type=jax.ShapeDtypeStruct((L * 4 * NW * 8 * IDX_MINOR,),
                                      jnp.float32),
        mesh=mesh,
        scratch_types=[
            pltpu.VMEM((CL * IDX_MINOR,), jnp.int32),
            pltpu.VMEM((CL * IDX_MINOR,), jnp.int32),
            pltpu.VMEM((CL * IDX_MINOR, D), jnp.float32),
            pltpu.VMEM((CL * IDX_MINOR, D), jnp.float32),
            pltpu.VMEM((NLINE * 8 * IDX_MINOR,), jnp.float32),
            pltpu.VMEM((NLINE * 8 * IDX_MINOR,), jnp.float32),
            pltpu.SemaphoreType.DMA,
            pltpu.SemaphoreType.DMA,
            pltpu.SemaphoreType.DMA,
            pltpu.SemaphoreType.DMA,
        ],
        compiler_params=pltpu.CompilerParams(needs_layout_passes=False,
                                             use_tc_tiling_on_sc=False),
    )
    out4 = k(table_rm, idxt)
    out = (out4.reshape(L, 4, NW, 8, IDX_MINOR)
           .transpose(2, 4, 0, 1, 3)
           .reshape(B, L, D))
    return out
